# Initial kernel scaffold; baseline (speedup 1.0000x reference)
import jax
import jax.numpy as jnp
from jax.experimental import pallas as pl


def kernel(x, edge_index, batch_idx, W1, b1, g1, bt1, W2, b2, g2, bt2, W3, b3, g3, bt3, Wc, bc):
    raise NotImplementedError("write your pallas kernel here")


# speedup vs baseline: 6.5033x; 6.5033x over previous
import functools

import jax
import jax.numpy as jnp
from jax import lax
from jax.experimental import pallas as pl
from jax.experimental.pallas import tpu as pltpu
from jax.experimental.pallas import tpu_sc as plsc

N = 10000
E = 160000
NG = 64
NC, NS, LANES = 2, 16, 16

BN = 2000
K = 80
EPT = E // NS
NP = 10240
RPT = NP // NS
ZB = 32

_mesh = lambda: plsc.VectorSubcoreMesh(core_axis_name="c", subcore_axis_name="s")


def _deg_kernel(dst_hbm, out_hbm, idxv, onesv, zbuf, acc,):
    cid = lax.axis_index("c")
    sid = lax.axis_index("s")
    KD = 40

    for i in range(KD):
        for j in range(128 // LANES):
            onesv[i, pl.ds(LANES * j, LANES)] = jnp.full((LANES,), 1.0,
                                                         jnp.float32)
    for i in range(ZB):
        for j in range(128 // LANES):
            zbuf[i, pl.ds(LANES * j, LANES)] = jnp.zeros((LANES,),
                                                         jnp.float32)

    def _zero(t, _):
        pltpu.sync_copy(zbuf, acc.at[pl.ds(sid * RPT + t * ZB, ZB)])
        return 0
    lax.fori_loop(0, RPT // ZB, _zero, 0)
    plsc.subcore_barrier()

    def _count(t, _):
        base = cid * (E // NC) + sid * (E // (NC * NS)) + t * KD
        pltpu.sync_copy(dst_hbm.at[pl.ds(base, KD)], idxv)
        pltpu.sync_copy(onesv, acc.at[idxv], add=True)
        return 0
    lax.fori_loop(0, (E // (NC * NS)) // KD, _count, 0)
    plsc.subcore_barrier()

    pltpu.sync_copy(acc.at[pl.ds(sid * RPT, RPT)],
                    out_hbm.at[pl.ds(cid * NP + sid * RPT, RPT)])


def _sc_degree(dst):
    f = pl.kernel(
        _deg_kernel,
        out_type=jax.ShapeDtypeStruct((NC * NP, 128), jnp.float32),
        mesh=_mesh(),
        scratch_types=[
            pltpu.VMEM((40,), jnp.int32),
            pltpu.VMEM((40, 128), jnp.float32),
            pltpu.VMEM((ZB, 128), jnp.float32),
            pltpu.VMEM_SHARED((NP, 128), jnp.float32),
        ],
    )
    return f(dst)


def _spmm_body(D2, feat_split, KE, xs_hbm, src_hbm, dst_hbm, out_hbm,
               srcv, dstv, rows, zbuf, acc, sem):
    cid = lax.axis_index("c")
    sid = lax.axis_index("s")

    for i in range(ZB):
        for j in range(D2 // LANES):
            zbuf[i, pl.ds(LANES * j, LANES)] = jnp.zeros((LANES,), jnp.float32)

    def _zero(t, _):
        pltpu.sync_copy(zbuf, acc.at[pl.ds(sid * RPT + t * ZB, ZB)])
        return 0
    lax.fori_loop(0, RPT // ZB, _zero, 0)
    plsc.subcore_barrier()

    ept = E // NS if feat_split else E // (NC * NS)

    def _edges(t, _):
        if feat_split:
            base = sid * ept + t * KE
        else:
            base = cid * (E // NC) + sid * ept + t * KE
        pltpu.sync_copy(src_hbm.at[pl.ds(base, KE)], srcv)
        pltpu.sync_copy(dst_hbm.at[pl.ds(base, KE)], dstv)
        if feat_split:
            off = cid * N
            for j in range(KE // LANES):
                srcv[pl.ds(LANES * j, LANES)] = srcv[pl.ds(LANES * j, LANES)] + off
        pltpu.async_copy(xs_hbm.at[srcv], rows, sem).wait()
        pltpu.sync_copy(rows, acc.at[dstv], add=True)
        return 0
    lax.fori_loop(0, ept // KE, _edges, 0)
    plsc.subcore_barrier()

    pltpu.sync_copy(acc.at[pl.ds(sid * RPT, RPT)],
                    out_hbm.at[pl.ds(cid * NP + sid * RPT, RPT)])


def _sc_spmm(xs2, src, dst, D2, feat_split):
    KE = K if feat_split else 40
    f = pl.kernel(
        functools.partial(_spmm_body, D2, feat_split, KE),
        out_type=jax.ShapeDtypeStruct((NC * NP, D2), jnp.float32),
        mesh=_mesh(),
        scratch_types=[
            pltpu.VMEM((KE,), jnp.int32),
            pltpu.VMEM((KE,), jnp.int32),
            pltpu.VMEM((KE, D2), jnp.float32),
            pltpu.VMEM((ZB, D2), jnp.float32),
            pltpu.VMEM_SHARED((NP, D2), jnp.float32),
            pltpu.SemaphoreType.DMA,
        ],
    )
    return f(xs2, src, dst)


def _dinv_block(cnt_ref):
    c = cnt_ref[0][:, 0:1] + cnt_ref[1][:, 0:1]
    return lax.rsqrt(jnp.maximum(1.0 + c, 1.0))


def _tc1_body(x_ref, w_ref, cnt_ref, out_ref):
    x = x_ref[...]
    x = jnp.nan_to_num(x, nan=0.0, posinf=1.0, neginf=-1.0)
    dinv = _dinv_block(cnt_ref)
    xs = jnp.dot(x, w_ref[...], preferred_element_type=jnp.float32,
                 precision=lax.Precision.HIGHEST) * dinv
    d2 = xs.shape[1] // 2
    out_ref[0] = xs[:, :d2]
    out_ref[1] = xs[:, d2:]


def _tc_layer_body(in_split, out_split, d_out,
                   agg_ref, xs_ref, cnt_ref, b_ref, g_ref, bt_ref, w_ref,
                   out_ref):
    if in_split:
        aggf = jnp.concatenate([agg_ref[0], agg_ref[1]], axis=1)
        xsf = jnp.concatenate([xs_ref[0], xs_ref[1]], axis=1)
    else:
        aggf = agg_ref[0] + agg_ref[1]
        xsf = xs_ref[...]
    dinv = _dinv_block(cnt_ref)
    conv = dinv * (aggf + xsf) + b_ref[...]
    mu = jnp.mean(conv, axis=1, keepdims=True)
    var = jnp.mean((conv - mu) ** 2, axis=1, keepdims=True)
    h = (conv - mu) * lax.rsqrt(var + 1e-5) * g_ref[...] + bt_ref[...]
    h = jnp.maximum(h, 0.0)
    xs = jnp.dot(h, w_ref[...], preferred_element_type=jnp.float32,
                 precision=lax.Precision.HIGHEST) * dinv
    if out_split:
        d2 = xs.shape[1] // 2
        out_ref[0] = xs[:, :d2]
        out_ref[1] = xs[:, d2:]
    elif d_out < 128:
        out_ref[...] = jnp.concatenate(
            [xs, jnp.zeros((xs.shape[0], 128 - d_out), jnp.float32)], axis=1)
    else:
        out_ref[...] = xs


def _tc_pool_body(agg_ref, xs_ref, cnt_ref, b_ref, g_ref, bt_ref, batch_ref,
                  psum_ref, pcnt_ref):
    @pl.when(pl.program_id(0) == 0)
    def _():
        psum_ref[...] = jnp.zeros_like(psum_ref)
        pcnt_ref[...] = jnp.zeros_like(pcnt_ref)

    aggf = (agg_ref[0] + agg_ref[1])[:, :64]
    xsf = xs_ref[...][:, :64]
    dinv = _dinv_block(cnt_ref)
    conv = dinv * (aggf + xsf) + b_ref[...]
    mu = jnp.mean(conv, axis=1, keepdims=True)
    var = jnp.mean((conv - mu) ** 2, axis=1, keepdims=True)
    h = (conv - mu) * lax.rsqrt(var + 1e-5) * g_ref[...] + bt_ref[...]
    h = jnp.maximum(h, 0.0)

    b = batch_ref[...]
    oh = (b == lax.broadcasted_iota(jnp.int32, (b.shape[0], NG), 1))
    oh = oh.astype(jnp.float32)
    dn = (((0,), (0,)), ((), ()))
    psum_ref[...] += lax.dot_general(oh, h, dn,
                                     preferred_element_type=jnp.float32,
                 precision=lax.Precision.HIGHEST)
    ones = jnp.ones((b.shape[0], 1), jnp.float32)
    pcnt_ref[...] += lax.dot_general(oh, ones, dn,
                                     preferred_element_type=jnp.float32,
                 precision=lax.Precision.HIGHEST)


def _tc_head_body(psum_ref, pcnt_ref, wc_ref, bc_ref, out_ref):
    pooled = psum_ref[...] / jnp.maximum(pcnt_ref[...], 1.0)
    out = jnp.dot(pooled, wc_ref[...], preferred_element_type=jnp.float32,
                 precision=lax.Precision.HIGHEST)
    out = out + bc_ref[...]
    out_ref[...] = jnp.clip(out, -10.0, 10.0)


def _row_spec(d2):
    return pl.BlockSpec((2, BN, d2), lambda i: (0, i, 0))


_CNT_SPEC = None


def _vec_spec(d):
    return pl.BlockSpec((1, d), lambda i: (0, 0))


def _tc1(x, w1, counts):
    return pl.pallas_call(
        _tc1_body,
        grid=(N // BN,),
        in_specs=[
            pl.BlockSpec((BN, 256), lambda i: (i, 0)),
            pl.BlockSpec((256, 256), lambda i: (0, 0)),
            pl.BlockSpec((2, BN, LANES), lambda i: (0, i, 0)),
        ],
        out_specs=_row_spec(128),
        out_shape=jax.ShapeDtypeStruct((2, N, 128), jnp.float32),
    )(x, w1, counts)


def _tc_layer(agg, xs, counts, b, g, bt, w, d_in, d_out, in_split, out_split):
    if in_split:
        agg_spec = _row_spec(d_in // 2)
        xs_spec = _row_spec(d_in // 2)
    else:
        agg_spec = _row_spec(128)
        xs_spec = pl.BlockSpec((BN, 128), lambda i: (i, 0))
    if out_split:
        out_spec = _row_spec(d_out // 2)
        out_shape = jax.ShapeDtypeStruct((2, N, d_out // 2), jnp.float32)
    else:
        out_spec = pl.BlockSpec((BN, 128), lambda i: (i, 0))
        out_shape = jax.ShapeDtypeStruct((N, 128), jnp.float32)
    return pl.pallas_call(
        functools.partial(_tc_layer_body, in_split, out_split, d_out),
        grid=(N // BN,),
        in_specs=[
            agg_spec,
            xs_spec,
            pl.BlockSpec((2, BN, LANES), lambda i: (0, i, 0)),
            _vec_spec(d_in),
            _vec_spec(d_in),
            _vec_spec(d_in),
            pl.BlockSpec((d_in, d_out), lambda i: (0, 0)),
        ],
        out_specs=out_spec,
        out_shape=out_shape,
    )(agg, xs, counts, b, g, bt, w)


def _tc_pool(agg, xs, counts, b, g, bt, batch2):
    return pl.pallas_call(
        _tc_pool_body,
        grid=(N // BN,),
        in_specs=[
            _row_spec(128),
            pl.BlockSpec((BN, 128), lambda i: (i, 0)),
            pl.BlockSpec((2, BN, LANES), lambda i: (0, i, 0)),
            _vec_spec(64),
            _vec_spec(64),
            _vec_spec(64),
            pl.BlockSpec((BN, 1), lambda i: (i, 0)),
        ],
        out_specs=[
            pl.BlockSpec((NG, 64), lambda i: (0, 0)),
            pl.BlockSpec((NG, 1), lambda i: (0, 0)),
        ],
        out_shape=[
            jax.ShapeDtypeStruct((NG, 64), jnp.float32),
            jax.ShapeDtypeStruct((NG, 1), jnp.float32),
        ],
    )(agg, xs, counts, b, g, bt, batch2)


def _tc_head(psum, pcnt, wc, bc):
    return pl.pallas_call(
        _tc_head_body,
        in_specs=[
            pl.BlockSpec((NG, 64), lambda: (0, 0)),
            pl.BlockSpec((NG, 1), lambda: (0, 0)),
            pl.BlockSpec((64, 5), lambda: (0, 0)),
            pl.BlockSpec((1, 5), lambda: (0, 0)),
        ],
        out_specs=pl.BlockSpec((NG, 5), lambda: (0, 0)),
        out_shape=jax.ShapeDtypeStruct((NG, 5), jnp.float32),
    )(psum, pcnt, wc, bc)


def kernel(x, edge_index, batch_idx, W1, b1, g1, bt1, W2, b2, g2, bt2,
           W3, b3, g3, bt3, Wc, bc):
    src = edge_index[0]
    dst = edge_index[1]

    counts = _sc_degree(dst).reshape(2, NP, 128)[:, :N, :LANES]

    xs1 = _tc1(x, W1, counts)
    agg1 = _sc_spmm(xs1.reshape(2 * N, 128), src, dst, 128,
                    True).reshape(2, NP, 128)[:, :N, :]

    xs2 = _tc_layer(agg1, xs1, counts, b1.reshape(1, -1), g1.reshape(1, -1),
                    bt1.reshape(1, -1), W2, 256, 128,
                    in_split=True, out_split=False)
    agg2 = _sc_spmm(xs2, src, dst, 128,
                    False).reshape(2, NP, 128)[:, :N, :]

    xs3 = _tc_layer(agg2, xs2, counts, b2.reshape(1, -1), g2.reshape(1, -1),
                    bt2.reshape(1, -1), W3, 128, 64,
                    in_split=False, out_split=False)
    agg3 = _sc_spmm(xs3, src, dst, 128,
                    False).reshape(2, NP, 128)[:, :N, :]

    psum, pcnt = _tc_pool(agg3, xs3, counts, b3.reshape(1, -1),
                          g3.reshape(1, -1), bt3.reshape(1, -1),
                          batch_idx.reshape(-1, 1))
    return _tc_head(psum, pcnt, Wc, bc.reshape(1, -1))


# speedup vs baseline: 11.6446x; 1.7906x over previous
import functools

import jax
import jax.numpy as jnp
from jax import lax
from jax.experimental import pallas as pl
from jax.experimental.pallas import tpu as pltpu
from jax.experimental.pallas import tpu_sc as plsc

N = 10000
E = 160000
NG = 64
NC, NS, LANES = 2, 16, 16

BN = 2000
K = 80
EPT = E // NS
NP = 10240
RPT = NP // NS
ZB = 32

_mesh = lambda: plsc.VectorSubcoreMesh(core_axis_name="c", subcore_axis_name="s")


def _deg_kernel(dst_hbm, out_hbm, idxv, onesv, zbuf, acc,):
    cid = lax.axis_index("c")
    sid = lax.axis_index("s")
    KD = 40

    for i in range(KD):
        for j in range(128 // LANES):
            onesv[i, pl.ds(LANES * j, LANES)] = jnp.full((LANES,), 1.0,
                                                         jnp.float32)
    for i in range(ZB):
        for j in range(128 // LANES):
            zbuf[i, pl.ds(LANES * j, LANES)] = jnp.zeros((LANES,),
                                                         jnp.float32)

    def _zero(t, _):
        pltpu.sync_copy(zbuf, acc.at[pl.ds(sid * RPT + t * ZB, ZB)])
        return 0
    lax.fori_loop(0, RPT // ZB, _zero, 0)
    plsc.subcore_barrier()

    def _count(t, _):
        base = cid * (E // NC) + sid * (E // (NC * NS)) + t * KD
        pltpu.sync_copy(dst_hbm.at[pl.ds(base, KD)], idxv)
        pltpu.sync_copy(onesv, acc.at[idxv], add=True)
        return 0
    lax.fori_loop(0, (E // (NC * NS)) // KD, _count, 0)
    plsc.subcore_barrier()

    pltpu.sync_copy(acc.at[pl.ds(sid * RPT, RPT)],
                    out_hbm.at[pl.ds(cid * NP + sid * RPT, RPT)])


def _sc_degree(dst):
    f = pl.kernel(
        _deg_kernel,
        out_type=jax.ShapeDtypeStruct((NC * NP, 128), jnp.float32),
        mesh=_mesh(),
        scratch_types=[
            pltpu.VMEM((40,), jnp.int32),
            pltpu.VMEM((40, 128), jnp.float32),
            pltpu.VMEM((ZB, 128), jnp.float32),
            pltpu.VMEM_SHARED((NP, 128), jnp.float32),
        ],
    )
    return f(dst)


def _spmm_body(D2, feat_split, KE, xs_hbm, src_hbm, dst_hbm, out_hbm,
               srcv0, srcv1, dstv0, dstv1, rows0, rows1, zbuf, acc,
               semi0, semi1, semg0, semg1):
    cid = lax.axis_index("c")
    sid = lax.axis_index("s")

    for i in range(ZB):
        for j in range(D2 // LANES):
            zbuf[i, pl.ds(LANES * j, LANES)] = jnp.zeros((LANES,), jnp.float32)

    def _zero(t, _):
        pltpu.sync_copy(zbuf, acc.at[pl.ds(sid * RPT + t * ZB, ZB)])
        return 0
    lax.fori_loop(0, RPT // ZB, _zero, 0)
    plsc.subcore_barrier()

    ept = E // NS if feat_split else E // (NC * NS)
    nc = ept // KE
    srcv = (srcv0, srcv1)
    dstv = (dstv0, dstv1)
    rows = (rows0, rows1)
    semi = (semi0, semi1)
    semg = (semg0, semg1)

    def _base(t):
        if feat_split:
            return sid * ept + t * KE
        return cid * (E // NC) + sid * ept + t * KE

    def _start_idx(b, t):
        pltpu.async_copy(src_hbm.at[pl.ds(_base(t), KE)], srcv[b], semi[b])
        pltpu.async_copy(dst_hbm.at[pl.ds(_base(t), KE)], dstv[b], semi[b])

    def _wait_idx(b, t):
        pltpu.make_async_copy(src_hbm.at[pl.ds(_base(t), KE)], srcv[b],
                              semi[b]).wait()
        pltpu.make_async_copy(dst_hbm.at[pl.ds(_base(t), KE)], dstv[b],
                              semi[b]).wait()

    def _start_gather(b):
        if feat_split:
            off = cid * N
            for j in range(KE // LANES):
                srcv[b][pl.ds(LANES * j, LANES)] = (
                    srcv[b][pl.ds(LANES * j, LANES)] + off)
        pltpu.async_copy(xs_hbm.at[srcv[b]], rows[b], semg[b])

    def _finish(b):
        pltpu.make_async_copy(xs_hbm.at[srcv[b]], rows[b], semg[b]).wait()
        pltpu.sync_copy(rows[b], acc.at[dstv[b]], add=True)

    _start_idx(0, 0)
    _wait_idx(0, 0)
    _start_gather(0)
    _start_idx(1, 1)

    def _steady(m, _):
        for ph in (1, 0):
            t = 2 * m + (1 if ph == 1 else 2)
            cur, prv = ph, 1 - ph
            _wait_idx(cur, t)
            _start_gather(cur)
            _finish(prv)

            @pl.when(t + 1 < nc)
            def _():
                _start_idx(prv, t + 1)
        return 0
    lax.fori_loop(0, (nc - 1) // 2, _steady, 0)
    _finish(0 if (nc % 2) == 1 else 1)
    plsc.subcore_barrier()

    pltpu.sync_copy(acc.at[pl.ds(sid * RPT, RPT)],
                    out_hbm.at[pl.ds(cid * NP + sid * RPT, RPT)])


def _sc_spmm(xs2, src, dst, D2, feat_split):
    KE = K if feat_split else 40
    f = pl.kernel(
        functools.partial(_spmm_body, D2, feat_split, KE),
        out_type=jax.ShapeDtypeStruct((NC * NP, D2), jnp.float32),
        mesh=_mesh(),
        scratch_types=[
            pltpu.VMEM((KE,), jnp.int32),
            pltpu.VMEM((KE,), jnp.int32),
            pltpu.VMEM((KE,), jnp.int32),
            pltpu.VMEM((KE,), jnp.int32),
            pltpu.VMEM((KE, D2), jnp.float32),
            pltpu.VMEM((KE, D2), jnp.float32),
            pltpu.VMEM((ZB, D2), jnp.float32),
            pltpu.VMEM_SHARED((NP, D2), jnp.float32),
            pltpu.SemaphoreType.DMA,
            pltpu.SemaphoreType.DMA,
            pltpu.SemaphoreType.DMA,
            pltpu.SemaphoreType.DMA,
        ],
    )
    return f(xs2, src, dst)


def _bdot(a, b):
    return jnp.dot(a.astype(jnp.bfloat16), b.astype(jnp.bfloat16),
                   preferred_element_type=jnp.float32)


def _dinv_block(cnt_ref):
    c = cnt_ref[0][:, 0:1] + cnt_ref[1][:, 0:1]
    return lax.rsqrt(jnp.maximum(1.0 + c, 1.0))


def _tc1_body(x_ref, w_ref, cnt_ref, out_ref):
    x = x_ref[...]
    x = jnp.nan_to_num(x, nan=0.0, posinf=1.0, neginf=-1.0)
    dinv = _dinv_block(cnt_ref)
    xs = _bdot(x, w_ref[...]) * dinv
    d2 = xs.shape[1] // 2
    out_ref[0] = xs[:, :d2]
    out_ref[1] = xs[:, d2:]


def _tc_layer_body(in_split, out_split, d_out,
                   agg_ref, xs_ref, cnt_ref, b_ref, g_ref, bt_ref, w_ref,
                   out_ref):
    if in_split:
        aggf = jnp.concatenate([agg_ref[0], agg_ref[1]], axis=1)
        xsf = jnp.concatenate([xs_ref[0], xs_ref[1]], axis=1)
    else:
        aggf = agg_ref[0] + agg_ref[1]
        xsf = xs_ref[...]
    dinv = _dinv_block(cnt_ref)
    conv = dinv * (aggf + xsf) + b_ref[...]
    mu = jnp.mean(conv, axis=1, keepdims=True)
    var = jnp.mean((conv - mu) ** 2, axis=1, keepdims=True)
    h = (conv - mu) * lax.rsqrt(var + 1e-5) * g_ref[...] + bt_ref[...]
    h = jnp.maximum(h, 0.0)
    xs = _bdot(h, w_ref[...]) * dinv
    if out_split:
        d2 = xs.shape[1] // 2
        out_ref[0] = xs[:, :d2]
        out_ref[1] = xs[:, d2:]
    elif d_out < 128:
        out_ref[...] = jnp.concatenate(
            [xs, jnp.zeros((xs.shape[0], 128 - d_out), jnp.float32)], axis=1)
    else:
        out_ref[...] = xs


def _tc_pool_body(agg_ref, xs_ref, cnt_ref, b_ref, g_ref, bt_ref, batch_ref,
                  psum_ref, pcnt_ref):
    @pl.when(pl.program_id(0) == 0)
    def _():
        psum_ref[...] = jnp.zeros_like(psum_ref)
        pcnt_ref[...] = jnp.zeros_like(pcnt_ref)

    aggf = (agg_ref[0] + agg_ref[1])[:, :64]
    xsf = xs_ref[...][:, :64]
    dinv = _dinv_block(cnt_ref)
    conv = dinv * (aggf + xsf) + b_ref[...]
    mu = jnp.mean(conv, axis=1, keepdims=True)
    var = jnp.mean((conv - mu) ** 2, axis=1, keepdims=True)
    h = (conv - mu) * lax.rsqrt(var + 1e-5) * g_ref[...] + bt_ref[...]
    h = jnp.maximum(h, 0.0)

    b = batch_ref[...]
    oh = (b == lax.broadcasted_iota(jnp.int32, (b.shape[0], NG), 1))
    oh = oh.astype(jnp.float32)
    dn = (((0,), (0,)), ((), ()))
    psum_ref[...] += lax.dot_general(oh, h, dn,
                                     preferred_element_type=jnp.float32,
                 precision=lax.Precision.HIGHEST)
    ones = jnp.ones((b.shape[0], 1), jnp.float32)
    pcnt_ref[...] += lax.dot_general(oh, ones, dn,
                                     preferred_element_type=jnp.float32,
                 precision=lax.Precision.HIGHEST)


def _tc_head_body(psum_ref, pcnt_ref, wc_ref, bc_ref, out_ref):
    pooled = psum_ref[...] / jnp.maximum(pcnt_ref[...], 1.0)
    out = _bdot(pooled, wc_ref[...])
    out = out + bc_ref[...]
    out_ref[...] = jnp.clip(out, -10.0, 10.0)


def _row_spec(d2):
    return pl.BlockSpec((2, BN, d2), lambda i: (0, i, 0))


_CNT_SPEC = None


def _vec_spec(d):
    return pl.BlockSpec((1, d), lambda i: (0, 0))


def _tc1(x, w1, counts):
    return pl.pallas_call(
        _tc1_body,
        grid=(N // BN,),
        in_specs=[
            pl.BlockSpec((BN, 256), lambda i: (i, 0)),
            pl.BlockSpec((256, 256), lambda i: (0, 0)),
            pl.BlockSpec((2, BN, LANES), lambda i: (0, i, 0)),
        ],
        out_specs=_row_spec(128),
        out_shape=jax.ShapeDtypeStruct((2, N, 128), jnp.float32),
    )(x, w1, counts)


def _tc_layer(agg, xs, counts, b, g, bt, w, d_in, d_out, in_split, out_split):
    if in_split:
        agg_spec = _row_spec(d_in // 2)
        xs_spec = _row_spec(d_in // 2)
    else:
        agg_spec = _row_spec(128)
        xs_spec = pl.BlockSpec((BN, 128), lambda i: (i, 0))
    if out_split:
        out_spec = _row_spec(d_out // 2)
        out_shape = jax.ShapeDtypeStruct((2, N, d_out // 2), jnp.float32)
    else:
        out_spec = pl.BlockSpec((BN, 128), lambda i: (i, 0))
        out_shape = jax.ShapeDtypeStruct((N, 128), jnp.float32)
    return pl.pallas_call(
        functools.partial(_tc_layer_body, in_split, out_split, d_out),
        grid=(N // BN,),
        in_specs=[
            agg_spec,
            xs_spec,
            pl.BlockSpec((2, BN, LANES), lambda i: (0, i, 0)),
            _vec_spec(d_in),
            _vec_spec(d_in),
            _vec_spec(d_in),
            pl.BlockSpec((d_in, d_out), lambda i: (0, 0)),
        ],
        out_specs=out_spec,
        out_shape=out_shape,
    )(agg, xs, counts, b, g, bt, w)


def _tc_pool(agg, xs, counts, b, g, bt, batch2):
    return pl.pallas_call(
        _tc_pool_body,
        grid=(N // BN,),
        in_specs=[
            _row_spec(128),
            pl.BlockSpec((BN, 128), lambda i: (i, 0)),
            pl.BlockSpec((2, BN, LANES), lambda i: (0, i, 0)),
            _vec_spec(64),
            _vec_spec(64),
            _vec_spec(64),
            pl.BlockSpec((BN, 1), lambda i: (i, 0)),
        ],
        out_specs=[
            pl.BlockSpec((NG, 64), lambda i: (0, 0)),
            pl.BlockSpec((NG, 1), lambda i: (0, 0)),
        ],
        out_shape=[
            jax.ShapeDtypeStruct((NG, 64), jnp.float32),
            jax.ShapeDtypeStruct((NG, 1), jnp.float32),
        ],
    )(agg, xs, counts, b, g, bt, batch2)


def _tc_head(psum, pcnt, wc, bc):
    return pl.pallas_call(
        _tc_head_body,
        in_specs=[
            pl.BlockSpec((NG, 64), lambda: (0, 0)),
            pl.BlockSpec((NG, 1), lambda: (0, 0)),
            pl.BlockSpec((64, 5), lambda: (0, 0)),
            pl.BlockSpec((1, 5), lambda: (0, 0)),
        ],
        out_specs=pl.BlockSpec((NG, 5), lambda: (0, 0)),
        out_shape=jax.ShapeDtypeStruct((NG, 5), jnp.float32),
    )(psum, pcnt, wc, bc)


def kernel(x, edge_index, batch_idx, W1, b1, g1, bt1, W2, b2, g2, bt2,
           W3, b3, g3, bt3, Wc, bc):
    src = edge_index[0]
    dst = edge_index[1]

    counts = _sc_degree(dst).reshape(2, NP, 128)[:, :N, :LANES]

    xs1 = _tc1(x, W1, counts)
    agg1 = _sc_spmm(xs1.reshape(2 * N, 128), src, dst, 128,
                    True).reshape(2, NP, 128)[:, :N, :]

    xs2 = _tc_layer(agg1, xs1, counts, b1.reshape(1, -1), g1.reshape(1, -1),
                    bt1.reshape(1, -1), W2, 256, 128,
                    in_split=True, out_split=False)
    agg2 = _sc_spmm(xs2, src, dst, 128,
                    False).reshape(2, NP, 128)[:, :N, :]

    xs3 = _tc_layer(agg2, xs2, counts, b2.reshape(1, -1), g2.reshape(1, -1),
                    bt2.reshape(1, -1), W3, 128, 64,
                    in_split=False, out_split=False)
    agg3 = _sc_spmm(xs3, src, dst, 128,
                    False).reshape(2, NP, 128)[:, :N, :]

    psum, pcnt = _tc_pool(agg3, xs3, counts, b3.reshape(1, -1),
                          g3.reshape(1, -1), bt3.reshape(1, -1),
                          batch_idx.reshape(-1, 1))
    return _tc_head(psum, pcnt, Wc, bc.reshape(1, -1))
